# baseline (speedup 1.0000x reference)
import jax
import jax.numpy as jnp
from jax import lax
from jax.experimental import pallas as pl
from jax.experimental.pallas import tpu as pltpu
from jax.experimental.pallas import tpu_sc as plsc

_B, _V, _F = 1024, 6890, 13776
_PV, _PF = 1000, 2000
_L = 16
_NW = 32
_BPW = _B // _NW
_ROW = _V * 3
_SLOT = _ROW + 2
_MEAN_FULL = _PV // _L
_MEAN_TAIL = _PV - _MEAN_FULL * _L
_VIDS_PAD = 1024
_FACE_ITERS = _PF // _L


def _sc_body(verts_hbm, faces_hbm, vids_hbm, fid_hbm, out_hbm,
             vert_buf, faces_v, fid_v, tri_v, vids_v, res_v, sem0, sem1):
    cid = lax.axis_index("c")
    sid = lax.axis_index("s")
    wid = sid * 2 + cid
    base = wid * _BPW

    lanes = lax.iota(jnp.int32, _L)
    zf = jnp.zeros((_L,), jnp.float32)
    zi = jnp.zeros((_L,), jnp.int32)

    def _start(slot, sem, b):
        pltpu.make_async_copy(
            verts_hbm.at[b], vert_buf.at[pl.ds(slot * _SLOT, _ROW)], sem
        ).start()

    def _wait(slot, sem):
        pltpu.make_async_copy(
            verts_hbm.at[0], vert_buf.at[pl.ds(slot * _SLOT, _ROW)], sem
        ).wait()

    _start(0, sem0, base)

    pltpu.sync_copy(faces_hbm, faces_v)
    pltpu.sync_copy(fid_hbm, fid_v)
    pltpu.sync_copy(vids_hbm, vids_v.at[pl.ds(0, _PV)])
    vids_v[pl.ds(_PV, _L)] = zi
    vids_v[pl.ds(_PV + _L - 8, _L)] = zi

    def vids_body(k, _):
        v = vids_v[pl.ds(k * _L, _L)]
        vids_v[pl.ds(k * _L, _L)] = v * 3
        return 0

    lax.fori_loop(0, _VIDS_PAD // _L, vids_body, 0)

    def tri_body(j, _):
        f3 = fid_v[pl.ds(j * _L, _L)] * 3
        ia = plsc.load_gather(faces_v, [f3])
        ib = plsc.load_gather(faces_v, [f3 + 1])
        ic = plsc.load_gather(faces_v, [f3 + 2])
        tri_v[pl.ds(j * _L, _L)] = ia * 3
        tri_v[pl.ds(_PF + j * _L, _L)] = ib * 3
        tri_v[pl.ds(2 * _PF + j * _L, _L)] = ic * 3
        return 0

    lax.fori_loop(0, _FACE_ITERS, tri_body, 0)

    def batch_body(i, carry):
        res0, res1 = carry
        par = jnp.bitwise_and(i, 1)

        @pl.when(par == 0)
        def _():
            _wait(0, sem0)

        @pl.when(par == 1)
        def _():
            _wait(1, sem1)

        @pl.when(jnp.logical_and(i + 1 < _BPW, par == 0))
        def _():
            _start(1, sem1, base + i + 1)

        @pl.when(jnp.logical_and(i + 1 < _BPW, par == 1))
        def _():
            _start(0, sem0, base + i + 1)

        off = par * _SLOT

        def mean_body(k, c):
            sx, sy, sz = c
            ix = vids_v[pl.ds(k * _L, _L)] + off
            sx = sx + plsc.load_gather(vert_buf, [ix])
            sy = sy + plsc.load_gather(vert_buf, [ix + 1])
            sz = sz + plsc.load_gather(vert_buf, [ix + 2])
            return sx, sy, sz

        sx, sy, sz = lax.fori_loop(0, _MEAN_FULL, mean_body, (zf, zf, zf))
        ix = vids_v[pl.ds(_MEAN_FULL * _L, _L)] + off
        tmask = lanes < _MEAN_TAIL
        sx = sx + jnp.where(tmask, plsc.load_gather(vert_buf, [ix]), 0.0)
        sy = sy + jnp.where(tmask, plsc.load_gather(vert_buf, [ix + 1]), 0.0)
        sz = sz + jnp.where(tmask, plsc.load_gather(vert_buf, [ix + 2]), 0.0)
        mx = jnp.sum(sx) * (1.0 / _PV)
        my = jnp.sum(sy) * (1.0 / _PV)
        mz = jnp.sum(sz) * (1.0 / _PV)

        def face_body(j, acc):
            ja = tri_v[pl.ds(j * _L, _L)] + off
            jb = tri_v[pl.ds(_PF + j * _L, _L)] + off
            jc = tri_v[pl.ds(2 * _PF + j * _L, _L)] + off
            ax = plsc.load_gather(vert_buf, [ja]) - mx
            ay = plsc.load_gather(vert_buf, [ja + 1]) - my
            az = plsc.load_gather(vert_buf, [ja + 2]) - mz
            bx = plsc.load_gather(vert_buf, [jb]) - mx
            by = plsc.load_gather(vert_buf, [jb + 1]) - my
            bz = plsc.load_gather(vert_buf, [jb + 2]) - mz
            cx = plsc.load_gather(vert_buf, [jc]) - mx
            cy = plsc.load_gather(vert_buf, [jc + 1]) - my
            cz = plsc.load_gather(vert_buf, [jc + 2]) - mz
            det = (ax * (by * cz - bz * cy)
                   - ay * (bx * cz - bz * cx)
                   + az * (bx * cy - by * cx))
            return acc + det

        acc = lax.fori_loop(0, _FACE_ITERS, face_body, zf)
        vol = jnp.abs(jnp.sum(acc)) * (1.0 / 6.0)

        res0 = jnp.where(lanes == i, vol, res0)
        res1 = jnp.where(lanes == (i - _L), vol, res1)
        return res0, res1

    res0, res1 = lax.fori_loop(0, _BPW, batch_body, (zf, zf))
    res_v[pl.ds(0, _L)] = res0
    res_v[pl.ds(_L, _L)] = res1
    pltpu.sync_copy(res_v, out_hbm.at[pl.ds(base, _BPW)])


def kernel(vertices, faces, part_vids, part_fid):
    verts2 = vertices.reshape(_B, _ROW)
    faces32 = faces.astype(jnp.int32).reshape(_F * 3)
    vids32 = part_vids.astype(jnp.int32)
    fid32 = part_fid.astype(jnp.int32)
    mesh = plsc.VectorSubcoreMesh(core_axis_name="c", subcore_axis_name="s")
    run = pl.kernel(
        _sc_body,
        out_type=jax.ShapeDtypeStruct((_B,), jnp.float32),
        mesh=mesh,
        compiler_params=pltpu.CompilerParams(
            needs_layout_passes=False, use_tc_tiling_on_sc=False
        ),
        scratch_types=[
            pltpu.VMEM((2 * _SLOT,), jnp.float32),
            pltpu.VMEM((_F * 3,), jnp.int32),
            pltpu.VMEM((_PF,), jnp.int32),
            pltpu.VMEM((3 * _PF,), jnp.int32),
            pltpu.VMEM((_VIDS_PAD,), jnp.int32),
            pltpu.VMEM((_BPW,), jnp.float32),
            pltpu.SemaphoreType.DMA,
            pltpu.SemaphoreType.DMA,
        ],
    )
    return run(verts2, faces32, vids32, fid32)
